# baseline (speedup 1.0000x reference)
import functools

import jax
import jax.numpy as jnp
from jax import lax
from jax.experimental import pallas as pl
from jax.experimental.pallas import tpu as pltpu
from jax.experimental.pallas import tpu_sc as plsc

E = 64
K = 8
_T = 256

_NC = 2
_NS = 16
_NTILE = _NC * _NS
_L = 16

_NEG = float("-inf")


def _logits_body(x_ref, w_ref, b_ref, out_ref):
    lg = lax.dot_general(
        w_ref[...], x_ref[...], (((1,), (1,)), ((), ())),
        preferred_element_type=jnp.float32)
    out_ref[...] = (lg + b_ref[...])[None]


def _tc_logits(xf, W, b2, n):
    grid = n // _T
    tpb = n // _NTILE
    steps_per_tile = tpb // _T
    return pl.pallas_call(
        _logits_body,
        grid=(grid,),
        in_specs=[
            pl.BlockSpec((_T, xf.shape[1]), lambda i: (i, 0)),
            pl.BlockSpec((E, xf.shape[1]), lambda i: (0, 0)),
            pl.BlockSpec((E, 1), lambda i: (0, 0)),
        ],
        out_specs=pl.BlockSpec(
            (1, E, _T),
            lambda i: (i // steps_per_tile, 0, i % steps_per_tile)),
        out_shape=jax.ShapeDtypeStruct((_NTILE, E, tpb), jnp.float32),
        compiler_params=pltpu.CompilerParams(
            dimension_semantics=("parallel",),
        ),
    )(xf, W, b2)


def _sc_router_body(tpb, lg_hbm, w_hbm, mask_hbm, idx_hbm,
                    slab, maskv, wv, iv, sem):
    wid = lax.axis_index("s") * _NC + lax.axis_index("c")
    cp = pltpu.make_async_copy(lg_hbm.at[wid], slab, sem)
    cp.start()
    zero = jnp.zeros((_L,), jnp.float32)

    def zbody(r, c):
        for j in range(8):
            maskv[pl.ds(r * 8 * _L + j * _L, _L)] = zero
        return c
    lax.fori_loop(0, tpb * E // (8 * _L), zbody, 0)
    cp.wait()

    toki = lax.broadcasted_iota(jnp.int32, (_L,), 0)
    negv = jnp.full((_L,), _NEG, jnp.float32)
    zeroi = jnp.zeros((_L,), jnp.int32)

    def group(g, c):
        goff = g * _L
        tokrel = goff + toki
        ms, mis = [], []
        for k in range(K):
            def scan(eb, carry):
                m, mi = carry
                for j in range(8):
                    e = eb * 8 + j
                    v = slab[pl.ds(e * tpb + goff, _L)]
                    gt = v > m
                    m = jnp.where(gt, v, m)
                    mi = jnp.where(gt, jnp.full((_L,), e, jnp.int32), mi)
                return (m, mi)
            m, mi = lax.fori_loop(0, E // 8, scan, (negv, zeroi))
            ms.append(m)
            mis.append(mi)
            if k < K - 1:
                plsc.store_scatter(slab, [mi * tpb + tokrel], negv)
        es = [jnp.exp(m - ms[0]) for m in ms]
        s = es[0]
        for k in range(1, K):
            s = s + es[k]
        r = 1.0 / s
        for k in range(K):
            wk = es[k] * r
            wv[pl.ds(k * tpb + goff, _L)] = wk
            iv[pl.ds(k * tpb + goff, _L)] = mis[k]
            plsc.store_scatter(maskv, [tokrel * E + mis[k]], wk)
        return c
    lax.fori_loop(0, tpb // _L, group, 0)

    pltpu.sync_copy(wv, w_hbm.at[wid])
    pltpu.sync_copy(maskv, mask_hbm.at[wid])
    pltpu.sync_copy(iv, idx_hbm.at[wid])


def _sc_router(lg3, tpb):
    mesh = plsc.VectorSubcoreMesh(core_axis_name="c", subcore_axis_name="s")
    return pl.kernel(
        functools.partial(_sc_router_body, tpb),
        out_type=[
            jax.ShapeDtypeStruct((_NTILE, K * tpb), jnp.float32),
            jax.ShapeDtypeStruct((_NTILE, tpb * E), jnp.float32),
            jax.ShapeDtypeStruct((_NTILE, K * tpb), jnp.int32),
        ],
        mesh=mesh,
        compiler_params=pltpu.CompilerParams(needs_layout_passes=False),
        scratch_types=[
            pltpu.VMEM((E * tpb,), jnp.float32),
            pltpu.VMEM((tpb * E,), jnp.float32),
            pltpu.VMEM((K * tpb,), jnp.float32),
            pltpu.VMEM((K * tpb,), jnp.int32),
            pltpu.SemaphoreType.DMA,
        ],
    )(lg3.reshape(_NTILE, E * tpb))


@jax.jit
def kernel(x, W, b):
    B, S, D = x.shape
    n = B * S
    tpb = n // _NTILE
    xf = x.reshape(n, D)
    b2 = b.reshape(E, 1)
    lg3 = _tc_logits(xf, W, b2, n)
    w3, mask3, idx3 = _sc_router(lg3, tpb)
    w = w3.reshape(_NTILE, K, tpb).transpose(0, 2, 1).reshape(B, S, K)
    mask = mask3.reshape(B, S, E)
    idx = idx3.reshape(_NTILE, K, tpb).transpose(0, 2, 1).reshape(B, S, K)
    return (w, mask, idx)
